# baseline (speedup 1.0000x reference)
import functools

import jax
import jax.numpy as jnp
from jax import lax
from jax.experimental import pallas as pl
from jax.experimental.pallas import tpu as pltpu
from jax.experimental.pallas import tpu_sc as plsc

EMB = 64
SCALE = 1.0 / (EMB ** 0.5)
LANES = 16
NC, NS = 2, 16
NW = NC * NS
CH = 128


def _make_lookup(n_rows: int):
    per_w = n_rows // NW
    nch = per_w // CH

    mesh = plsc.VectorSubcoreMesh(core_axis_name="c", subcore_axis_name="s")

    @functools.partial(
        pl.kernel,
        mesh=mesh,
        out_type=jax.ShapeDtypeStruct((n_rows, EMB), jnp.float32),
        compiler_params=pltpu.CompilerParams(use_tc_tiling_on_sc=False),
        scratch_types=[
            pltpu.VMEM((nch, CH), jnp.int32),
            pltpu.VMEM((CH, EMB), jnp.float32),
            pltpu.SemaphoreType.DMA,
        ],
    )
    def lookup(x_hbm, tbl_hbm, out_hbm, idx_v, rows_v, gsem):
        cid = lax.axis_index("c")
        sid = lax.axis_index("s")
        wid = sid * NC + cid
        base = wid * per_w
        pltpu.sync_copy(x_hbm.at[wid], idx_v)

        def chunk(g, carry):
            pltpu.async_copy(tbl_hbm.at[idx_v.at[g]], rows_v, gsem).wait()

            def srow(r, c):
                for j in range(EMB // LANES):
                    sl = pl.ds(j * LANES, LANES)
                    rows_v[r, sl] = rows_v[r, sl] * SCALE
                return c

            lax.fori_loop(0, CH, srow, 0)
            pltpu.sync_copy(rows_v, out_hbm.at[pl.ds(base + g * CH, CH)])
            return carry

        lax.fori_loop(0, nch, chunk, 0)

    return lookup


def kernel(x, table):
    B, S = x.shape
    n = B * S
    xr = x.astype(jnp.int32).reshape(NW, n // NW // CH, CH)
    out = _make_lookup(n)(xr, table)
    return out.reshape(B, S, EMB)
